# baseline (speedup 1.0000x reference)
import jax
import jax.numpy as jnp
from jax import lax
from jax.experimental import pallas as pl
from jax.experimental.pallas import tpu as pltpu
from jax.experimental.pallas import tpu_sc as plsc

N_V = 1000
N_D = 64
BATCH = 4096
HIST = 200

NC = 2
NS = 16
NW = NC * NS
L = 16

B_TOTAL = BATCH * HIST
ROWS_PER_W = B_TOTAL // NW
GROUP = 256
N_GROUPS = ROWS_PER_W // GROUP
BLOCKS = GROUP // L
GROUP_WORDS = GROUP * N_D
CHUNKS_PER_GROUP = GROUP // 128

T_STRIDE = 8 * 32 * 8 * 128
CH_STRIDE = 32 * 8 * 128
BH_STRIDE = 8 * 128
SG_CH = CHUNKS_PER_GROUP * 8 * 128
SG_BH = 8 * 128


def _embed_body(idx_hbm, table_hbm, out_hbm, idx_v, table_v, rows_v, wsems):
  wid = lax.axis_index("s") * NC + lax.axis_index("c")
  p_base = wid * (ROWS_PER_W // 128)

  pltpu.sync_copy(table_hbm, table_v)
  pltpu.sync_copy(idx_hbm.at[pl.ds(wid * ROWS_PER_W, ROWS_PER_W)], idx_v)

  lanes = lax.iota(jnp.int32, L)
  diag = [(lanes + d) & (L - 1) for d in range(L)]
  sgoff = [((diag[d] >> 3) * SG_CH) + ((diag[d] & 7) * 128) + lanes
           for d in range(L)]

  def hbm_off(g, ch):
    p0 = p_base + g * CHUNKS_PER_GROUP
    t = p0 >> 5
    bh0 = p0 & 31
    return t * T_STRIDE + ch * CH_STRIDE + bh0 * BH_STRIDE

  def write_group(g, pg):
    for ch in range(8):
      pltpu.async_copy(
          rows_v.at[pl.ds(pg * GROUP_WORDS + ch * SG_CH, SG_CH)],
          out_hbm.at[pl.ds(hbm_off(g, ch), SG_CH)],
          wsems.at[pg])

  def wait_group(g, pg):
    for ch in range(8):
      pltpu.make_async_copy(
          rows_v.at[pl.ds(pg * GROUP_WORDS + ch * SG_CH, SG_CH)],
          out_hbm.at[pl.ds(hbm_off(g, ch), SG_CH)],
          wsems.at[pg]).wait()

  def expand_group(g, pg):
    pg_words = pg * GROUP_WORDS

    @plsc.parallel_loop(0, BLOCKS, unroll=8)
    def _(i):
      v = idx_v[pl.ds(g * GROUP + i * L, L)]
      src_base = v * N_D
      dst_base = pg_words + (i >> 3) * SG_BH + (i & 7) * L
      for bc in range(N_D // L):
        for d in range(L):
          col = plsc.load_gather(table_v, [src_base + (diag[d] + bc * L)])
          plsc.store_scatter(
              rows_v, [sgoff[d] + (dst_base + bc * 2 * SG_CH)], col)

  @pl.loop(0, N_GROUPS)
  def _(g):
    pg = lax.rem(g, 2)

    @pl.when(g >= 2)
    def _():
      wait_group(g - 2, pg)

    expand_group(g, pg)
    write_group(g, pg)

  for g in (N_GROUPS - 2, N_GROUPS - 1):
    wait_group(g, g % 2)


@jax.jit
def kernel(input_, W):
  idx_t = input_.T.reshape(B_TOTAL)
  table_flat = W.reshape(N_V * N_D)
  run = pl.kernel(
      _embed_body,
      out_type=jax.ShapeDtypeStruct((B_TOTAL * N_D,), jnp.float32),
      mesh=plsc.VectorSubcoreMesh(core_axis_name="c", subcore_axis_name="s"),
      scratch_types=[
          pltpu.VMEM((ROWS_PER_W,), jnp.int32),
          pltpu.VMEM((N_V * N_D,), jnp.float32),
          pltpu.VMEM((2 * GROUP_WORDS,), jnp.float32),
          pltpu.SemaphoreType.DMA((2,)),
      ],
      compiler_params=pltpu.CompilerParams(
          use_tc_tiling_on_sc=False, needs_layout_passes=False,
          disable_bounds_checks=True),
  )
  out = run(idx_t, table_flat)
  a = out.reshape(HIST, 8, 32, 8, 128)
  return a.transpose(2, 4, 0, 1, 3).reshape(BATCH, HIST, N_D)
